# baseline (speedup 1.0000x reference)
import functools

import jax
import jax.numpy as jnp
import numpy as np
from jax import lax
from jax.experimental import pallas as pl
from jax.experimental.pallas import tpu as pltpu
from jax.experimental.pallas import tpu_sc as plsc

_F32 = jnp.float32
_BIG = 1e30
_SQ = float(np.sqrt(np.float32(1.0 + 1e-5)))


def _bnrelu(A, g_ref, bt_ref):
    return jnp.maximum(A / _SQ * g_ref[...] + bt_ref[...], 0.0)


def _dot(a, b):
    return jnp.dot(a, b, preferred_element_type=_F32)


def _fps_body(x_ref, y_ref, z_ref, cx_ref, cy_ref, cz_ref, *, npoint):
    X = x_ref[...]
    Y = y_ref[...]
    Z = z_ref[...]
    B, N = X.shape
    jidx = lax.broadcasted_iota(jnp.int32, (B, N), 1)
    tlane = lax.broadcasted_iota(jnp.int32, (B, npoint), 1)

    def step(t, carry):
        dist, far, CX, CY, CZ = carry
        onehot = jidx == far
        cx = jnp.sum(jnp.where(onehot, X, 0.0), axis=1, keepdims=True)
        cy = jnp.sum(jnp.where(onehot, Y, 0.0), axis=1, keepdims=True)
        cz = jnp.sum(jnp.where(onehot, Z, 0.0), axis=1, keepdims=True)
        sel = tlane == t
        CX = jnp.where(sel, cx, CX)
        CY = jnp.where(sel, cy, CY)
        CZ = jnp.where(sel, cz, CZ)
        d = (X - cx) ** 2 + (Y - cy) ** 2 + (Z - cz) ** 2
        dist = jnp.minimum(dist, d)
        m = jnp.max(dist, axis=1, keepdims=True)
        far = jnp.min(jnp.where(dist == m, jidx, N), axis=1, keepdims=True)
        return dist, far, CX, CY, CZ

    init = (
        jnp.full((B, N), 1e10, _F32),
        jnp.zeros((B, 1), jnp.int32),
        jnp.zeros((B, npoint), _F32),
        jnp.zeros((B, npoint), _F32),
        jnp.zeros((B, npoint), _F32),
    )
    _, _, CX, CY, CZ = lax.fori_loop(0, npoint, step, init)
    cx_ref[...] = CX
    cy_ref[...] = CY
    cz_ref[...] = CZ


def _fps(X, Y, Z, npoint):
    B, N = X.shape
    out = jax.ShapeDtypeStruct((B, npoint), _F32)
    return pl.pallas_call(
        functools.partial(_fps_body, npoint=npoint),
        out_shape=(out, out, out),
    )(X, Y, Z)


def _bq_body(p3_ref, c_ref, idx_ref, *, ns, r2, N):
    b = pl.program_id(0)
    P3 = p3_ref[0]
    C = c_ref[...]
    SB = C.shape[0]
    pp2 = jnp.sum(P3 * P3, axis=0, keepdims=True)
    cc2 = jnp.sum(C * C, axis=1, keepdims=True)
    cross = _dot(C, P3)
    d2 = (cc2 + pp2) - 2.0 * cross
    jidx = lax.broadcasted_iota(jnp.int32, (SB, N), 1)
    klane = lax.broadcasted_iota(jnp.int32, (SB, ns), 1)
    work0 = jnp.where(d2 <= r2, jidx, N)

    def step(k, carry):
        work, IDX = carry
        cur = jnp.min(work, axis=1, keepdims=True)
        IDX = jnp.where(klane == k, cur, IDX)
        work = jnp.where(work == cur, N, work)
        return work, IDX

    _, IDX = lax.fori_loop(0, ns, step, (work0, jnp.full((SB, ns), N, jnp.int32)))
    first = IDX[:, 0:1]
    IDX = jnp.where(IDX < N, IDX, first)
    idx_ref[0] = IDX + b * N


def _ball_query(P3, Cf, S, ns, radius, sblk):
    B = P3.shape[0]
    N = P3.shape[2]
    nsc = S // sblk
    grid = (B, nsc)
    return pl.pallas_call(
        functools.partial(_bq_body, ns=ns, r2=np.float32(radius * radius), N=N),
        grid=grid,
        in_specs=[
            pl.BlockSpec((1, 3, N), lambda b, s: (b, 0, 0)),
            pl.BlockSpec((sblk, 3), lambda b, s: (b * nsc + s, 0)),
        ],
        out_specs=pl.BlockSpec((1, sblk, ns), lambda b, s: (b, s, 0)),
        out_shape=jax.ShapeDtypeStruct((B, S, ns), jnp.int32),
    )(P3, Cf)


def _sc_gather(table, idx, tc_tiling=True):
    V, D = table.shape
    Bi = idx.shape[0]
    info = plsc.get_sparse_core_info()
    NC, NS = info.num_cores, info.num_subcores
    NW = NC * NS
    CH = 128
    per_w = Bi // NW
    n_iter = per_w // CH
    mesh = plsc.VectorSubcoreMesh(core_axis_name="c", subcore_axis_name="s")

    @functools.partial(
        pl.kernel,
        mesh=mesh,
        compiler_params=pltpu.CompilerParams(use_tc_tiling_on_sc=tc_tiling),
        out_type=jax.ShapeDtypeStruct((Bi, D), _F32),
        scratch_types=[
            pltpu.VMEM((2, CH), jnp.int32),
            pltpu.VMEM((2, CH, D), _F32),
            pltpu.SemaphoreType.DMA,
            pltpu.SemaphoreType.DMA,
        ],
    )
    def k(table_hbm, idx_hbm, out_hbm, idx_v, rows_v, sem0, sem1):
        wid = lax.axis_index("s") * NC + lax.axis_index("c")
        base = wid * per_w
        sems = (sem0, sem1)

        def fetch(c, j):
            off = pl.multiple_of(base + c * CH, CH)
            pltpu.sync_copy(idx_hbm.at[pl.ds(off, CH)], idx_v.at[j])
            pltpu.async_copy(table_hbm.at[idx_v.at[j]], rows_v.at[j], sems[j])

        fetch(0, 0)

        def body(i, carry):
            for j in range(2):
                c = 2 * i + j

                @pl.when(c + 1 < n_iter)
                def _():
                    fetch(c + 1, 1 - j)

                pltpu.make_async_copy(
                    table_hbm.at[idx_v.at[j]], rows_v.at[j], sems[j]).wait()
                off = pl.multiple_of(base + c * CH, CH)
                pltpu.sync_copy(rows_v.at[j], out_hbm.at[pl.ds(off, CH)])
            return carry

        lax.fori_loop(0, n_iter // 2, body, 0)

    return k(table, idx)


def _sagroup_body(g_ref, c_ref, w1_ref, b1_ref, g1_ref, bt1_ref,
                  w2_ref, b2_ref, g2_ref, bt2_ref,
                  w3_ref, b3_ref, g3_ref, bt3_ref, out_ref,
                  *, ns, pw, co):
    G = g_ref[...]
    C = c_ref[...]
    RB = C.shape[0]
    crep = jnp.broadcast_to(C[:, None, :], (RB, ns, 3)).reshape(RB * ns, 3)
    rel = G[:, co:co + 3] - crep
    feat = jnp.concatenate([rel, G[:, 0:pw]], axis=1)
    A = _bnrelu(_dot(feat, w1_ref[...]) + b1_ref[...], g1_ref, bt1_ref)
    A = _bnrelu(_dot(A, w2_ref[...]) + b2_ref[...], g2_ref, bt2_ref)
    A = _bnrelu(_dot(A, w3_ref[...]) + b3_ref[...], g3_ref, bt3_ref)
    C3 = A.shape[1]
    out_ref[...] = jnp.max(A.reshape(RB, ns, C3), axis=1)


def _sa_group(G, C, layers, ns, rb, pw, co):
    S = C.shape[0]
    D = G.shape[1]
    (W1, b1, g1, bt1), (W2, b2, g2, bt2), (W3, b3, g3, bt3) = layers
    w1 = W1.T
    w2 = W2.T
    w3 = W3.T
    C3 = W3.shape[0]
    grid = (S // rb,)
    cst = lambda a: pl.BlockSpec(a.shape, lambda i: (0, 0))
    row = lambda a: pl.BlockSpec((1, a.shape[0]), lambda i: (0, 0))
    args = [G, C, w1, b1, g1, bt1, w2, b2, g2, bt2, w3, b3, g3, bt3]
    in_specs = [
        pl.BlockSpec((rb * ns, D), lambda i: (i, 0)),
        pl.BlockSpec((rb, 3), lambda i: (i, 0)),
        cst(w1), row(b1), row(g1), row(bt1),
        cst(w2), row(b2), row(g2), row(bt2),
        cst(w3), row(b3), row(g3), row(bt3),
    ]
    args = [a if a.ndim > 1 else a.reshape(1, -1) for a in args]
    return pl.pallas_call(
        functools.partial(_sagroup_body, ns=ns, pw=pw, co=co),
        grid=grid,
        in_specs=in_specs,
        out_specs=pl.BlockSpec((rb, C3), lambda i: (i, 0)),
        out_shape=jax.ShapeDtypeStruct((S, C3), _F32),
    )(*args)


def _sa3_body(x_ref, *rest, npts):
    A = x_ref[...]
    refs = rest[:-1]
    o_ref = rest[-1]
    for li in range(3):
        w_ref, b_ref, g_ref, bt_ref = refs[4 * li:4 * li + 4]
        A = _bnrelu(_dot(A, w_ref[...]) + b_ref[...], g_ref, bt_ref)
    R, Cc = A.shape
    o_ref[...] = jnp.max(A.reshape(R // npts, npts, Cc), axis=1)


def _sa3(x, layers, npts, rb):
    M = x.shape[0]
    C = layers[-1][0].shape[0]
    grid = (M // rb,)
    cst = lambda a: pl.BlockSpec(a.shape, lambda i: (0, 0))
    in_specs = [pl.BlockSpec((rb, x.shape[1]), lambda i: (i, 0))]
    args = [x]
    for (W, b, g, bt) in layers:
        wt = W.T
        in_specs += [cst(wt)] + [pl.BlockSpec((1, W.shape[0]), lambda i: (0, 0))] * 3
        args += [wt, b.reshape(1, -1), g.reshape(1, -1), bt.reshape(1, -1)]
    return pl.pallas_call(
        functools.partial(_sa3_body, npts=npts),
        grid=grid,
        in_specs=in_specs,
        out_specs=pl.BlockSpec((rb // npts, C), lambda i: (i, 0)),
        out_shape=jax.ShapeDtypeStruct((M // npts, C), _F32),
    )(*args)


def _fp3_body(x1_ref, l3_ref, w1_ref, b1_ref, g1_ref, bt1_ref,
              w2_ref, b2_ref, g2_ref, bt2_ref, o_ref):
    x1 = x1_ref[...]
    l3 = jnp.broadcast_to(l3_ref[0], (x1.shape[0], l3_ref.shape[2]))
    feat = jnp.concatenate([x1, l3], axis=1)
    A = _bnrelu(_dot(feat, w1_ref[...]) + b1_ref[...], g1_ref, bt1_ref)
    o_ref[...] = _bnrelu(_dot(A, w2_ref[...]) + b2_ref[...], g2_ref, bt2_ref)


def _fp3(x1, l3, layers, rb):
    (W1, b1, g1, bt1), (W2, b2, g2, bt2) = layers
    M = x1.shape[0]
    grid = (M // rb,)
    cst = lambda a: pl.BlockSpec(a.shape, lambda b: (0, 0))
    rowv = lambda n: pl.BlockSpec((1, n), lambda b: (0, 0))
    return pl.pallas_call(
        _fp3_body,
        grid=grid,
        in_specs=[
            pl.BlockSpec((rb, x1.shape[1]), lambda b: (b, 0)),
            pl.BlockSpec((1, 1, l3.shape[1]), lambda b: (b, 0, 0)),
            cst(W1.T), rowv(W1.shape[0]), rowv(W1.shape[0]), rowv(W1.shape[0]),
            cst(W2.T), rowv(W2.shape[0]), rowv(W2.shape[0]), rowv(W2.shape[0]),
        ],
        out_specs=pl.BlockSpec((rb, W2.shape[0]), lambda b: (b, 0)),
        out_shape=jax.ShapeDtypeStruct((M, W2.shape[0]), _F32),
    )(x1, l3.reshape(l3.shape[0], 1, l3.shape[1]), W1.T,
      b1.reshape(1, -1), g1.reshape(1, -1), bt1.reshape(1, -1),
      W2.T, b2.reshape(1, -1), g2.reshape(1, -1), bt2.reshape(1, -1))


def _fp_body(qc_ref, s3_ref, p1_ref, ps_ref,
             w1_ref, b1_ref, g1_ref, bt1_ref, rest_refs, o_ref,
             *, flags, dup_xyz):
    QC = qc_ref[...]
    S3 = s3_ref[0]
    NQ = QC.shape[0]
    NS = S3.shape[1]
    qq2 = jnp.sum(QC * QC, axis=1, keepdims=True)
    ss2 = jnp.sum(S3 * S3, axis=0, keepdims=True)
    cross = _dot(QC, S3)
    d2 = (qq2 + ss2) - 2.0 * cross
    jidx = lax.broadcasted_iota(jnp.int32, (NQ, NS), 1)

    work = d2
    ids = []
    ds = []
    for _ in range(3):
        m = jnp.min(work, axis=1, keepdims=True)
        i = jnp.min(jnp.where(work == m, jidx, NS), axis=1, keepdims=True)
        ids.append(i)
        ds.append(m)
        work = jnp.where(jidx == i, _BIG, work)
    r1 = 1.0 / (ds[0] + 1e-8)
    r2 = 1.0 / (ds[1] + 1e-8)
    r3 = 1.0 / (ds[2] + 1e-8)
    s = r1 + r2 + r3
    Wmat = (
        jnp.where(jidx == ids[0], r1 / s, 0.0)
        + jnp.where(jidx == ids[1], r2 / s, 0.0)
        + jnp.where(jidx == ids[2], r3 / s, 0.0)
    )
    interp = jnp.dot(Wmat, ps_ref[...], precision=lax.Precision.HIGHEST,
                     preferred_element_type=_F32)
    p1 = p1_ref[...]
    if dup_xyz:
        feat = jnp.concatenate([p1, p1, interp], axis=1)
    else:
        feat = jnp.concatenate([p1, interp], axis=1)
    A = _bnrelu(_dot(feat, w1_ref[...]) + b1_ref[...], g1_ref, bt1_ref)
    for li, (has_bn, has_relu) in enumerate(flags):
        w_ref, b_ref, g_ref, bt_ref = rest_refs[4 * li:4 * li + 4]
        A = _dot(A, w_ref[...]) + b_ref[...]
        if has_bn:
            A = A / _SQ * g_ref[...] + bt_ref[...]
        if has_relu:
            A = jnp.maximum(A, 0.0)
    o_ref[...] = A


def _fp_stage(QC, S3, P1, PS, layer1, rest, qb):
    B = S3.shape[0]
    Ns = S3.shape[2]
    Nq = QC.shape[0] // B
    C1 = P1.shape[1]
    W1, b1, g1, bt1 = layer1
    nq_blocks = Nq // qb
    grid = (B, nq_blocks)
    cst = lambda a: pl.BlockSpec(a.shape, lambda b, q: (0, 0))
    rowv = lambda n: pl.BlockSpec((1, n), lambda b, q: (0, 0))
    qspec = lambda c: pl.BlockSpec((qb, c), lambda b, q: (b * nq_blocks + q, 0))
    in_specs = [qspec(3),
                pl.BlockSpec((1, 3, Ns), lambda b, q: (b, 0, 0)),
                qspec(C1),
                pl.BlockSpec((Ns, PS.shape[1]), lambda b, q: (b, 0)),
                cst(W1.T), rowv(W1.shape[0]), rowv(W1.shape[0]),
                rowv(W1.shape[0])]
    args = [QC, S3, P1, PS, W1.T, b1.reshape(1, -1),
            g1.reshape(1, -1), bt1.reshape(1, -1)]
    flags = []
    for (W, bb, g, bt, has_bn, has_relu) in rest:
        wt = W.T
        in_specs += [cst(wt), rowv(W.shape[0]), rowv(W.shape[0]), rowv(W.shape[0])]
        args += [wt, bb.reshape(1, -1), g.reshape(1, -1), bt.reshape(1, -1)]
        flags.append((has_bn, has_relu))
    Cout = rest[-1][0].shape[0]

    def body(*refs):
        _fp_body(*refs[:8], refs[8:-1], refs[-1], flags=tuple(flags),
                 dup_xyz=(C1 == 3))

    return pl.pallas_call(
        body,
        grid=grid,
        in_specs=in_specs,
        out_specs=pl.BlockSpec((qb, Cout), lambda b, q: (b * nq_blocks + q, 0)),
        out_shape=jax.ShapeDtypeStruct((B * Nq, Cout), _F32),
    )(*args)


def _flat3(CX, CY, CZ):
    return jnp.stack([CX, CY, CZ], axis=-1).reshape(-1, 3)


def kernel(xyz, params):
    B, N, _ = xyz.shape
    X, Y, Z = xyz[..., 0], xyz[..., 1], xyz[..., 2]
    ones = lambda c: (jnp.zeros((c,), _F32), jnp.ones((c,), _F32),
                      jnp.zeros((c,), _F32))

    P3 = jnp.transpose(xyz, (0, 2, 1))

    CX1, CY1, CZ1 = _fps(X, Y, Z, 512)
    c1f = _flat3(CX1, CY1, CZ1)
    idx1 = _ball_query(P3, c1f, 512, 32, 0.2, 128)
    T1 = jnp.pad(xyz.reshape(-1, 3), ((0, 0), (0, 5)))
    G1 = _sc_gather(T1, idx1.reshape(-1), tc_tiling=False)
    l1p = _sa_group(G1, c1f, params['sa1'], 32, 128, pw=3, co=0)

    C1_3 = jnp.stack([CX1, CY1, CZ1], axis=1)
    CX2, CY2, CZ2 = _fps(CX1, CY1, CZ1, 128)
    c2f = _flat3(CX2, CY2, CZ2)
    idx2 = _ball_query(C1_3, c2f, 128, 64, 0.4, 128)
    T2 = jnp.concatenate(
        [l1p, c1f, jnp.zeros((c1f.shape[0], 125), _F32)], axis=1)
    G2 = _sc_gather(T2, idx2.reshape(-1))
    l2p = _sa_group(G2, c2f, params['sa2'], 64, 64, pw=128, co=128)

    x3 = jnp.concatenate([c2f, l2p], axis=1)
    l3 = _sa3(x3, params['sa3'], 128, 1024)

    l2p_new = _fp3(l2p, l3, params['fp3'], 128)

    (Wf1, bf1, gf1, btf1), (Wf2, bf2, gf2, btf2) = params['fp2']
    C2_3 = jnp.stack([CX2, CY2, CZ2], axis=1)
    l1p_new = _fp_stage(
        c1f, C2_3, l1p, l2p_new,
        (Wf1, bf1, gf1, btf1),
        [(Wf2, bf2, gf2, btf2, True, True)],
        qb=512,
    )

    fp1 = params['fp1']
    Wc1, bc1, gc1, btc1 = params['conv1']
    Wc2, bc2 = params['conv2']
    one1 = ones(Wc2.shape[0])
    rest = [
        (fp1[1][0], fp1[1][1], fp1[1][2], fp1[1][3], True, True),
        (fp1[2][0], fp1[2][1], fp1[2][2], fp1[2][3], True, True),
        (Wc1, bc1, gc1, btc1, True, True),
        (Wc2, bc2, one1[1], one1[0], False, False),
    ]
    out = _fp_stage(
        xyz.reshape(-1, 3), C1_3, xyz.reshape(-1, 3), l1p_new,
        (fp1[0][0], fp1[0][1], fp1[0][2], fp1[0][3]),
        rest,
        qb=1024,
    )
    return out.reshape(B, N, -1)


# speedup vs baseline: 1.0453x; 1.0453x over previous
import functools

import jax
import jax.numpy as jnp
import numpy as np
from jax import lax
from jax.experimental import pallas as pl
from jax.experimental.pallas import tpu as pltpu
from jax.experimental.pallas import tpu_sc as plsc

_F32 = jnp.float32
_BIG = 1e30
_SQ = float(np.sqrt(np.float32(1.0 + 1e-5)))


def _bnrelu(A, g_ref, bt_ref):
    return jnp.maximum(A / _SQ * g_ref[...] + bt_ref[...], 0.0)


def _dot(a, b):
    return jnp.dot(a, b, preferred_element_type=_F32)


def _fps_body(x_ref, y_ref, z_ref, cx_ref, cy_ref, cz_ref, *, npoint):
    X = x_ref[...]
    Y = y_ref[...]
    Z = z_ref[...]
    B, N = X.shape
    jidx = lax.broadcasted_iota(jnp.int32, (B, N), 1)
    tlane = lax.broadcasted_iota(jnp.int32, (B, npoint), 1)

    def step(t, carry):
        dist, far, CX, CY, CZ = carry
        onehot = jidx == far
        cx = jnp.sum(jnp.where(onehot, X, 0.0), axis=1, keepdims=True)
        cy = jnp.sum(jnp.where(onehot, Y, 0.0), axis=1, keepdims=True)
        cz = jnp.sum(jnp.where(onehot, Z, 0.0), axis=1, keepdims=True)
        sel = tlane == t
        CX = jnp.where(sel, cx, CX)
        CY = jnp.where(sel, cy, CY)
        CZ = jnp.where(sel, cz, CZ)
        d = (X - cx) ** 2 + (Y - cy) ** 2 + (Z - cz) ** 2
        dist = jnp.minimum(dist, d)
        m = jnp.max(dist, axis=1, keepdims=True)
        far = jnp.min(jnp.where(dist == m, jidx, N), axis=1, keepdims=True)
        return dist, far, CX, CY, CZ

    init = (
        jnp.full((B, N), 1e10, _F32),
        jnp.zeros((B, 1), jnp.int32),
        jnp.zeros((B, npoint), _F32),
        jnp.zeros((B, npoint), _F32),
        jnp.zeros((B, npoint), _F32),
    )
    _, _, CX, CY, CZ = lax.fori_loop(0, npoint, step, init)
    cx_ref[...] = CX
    cy_ref[...] = CY
    cz_ref[...] = CZ


def _fps(X, Y, Z, npoint):
    B, N = X.shape
    out = jax.ShapeDtypeStruct((B, npoint), _F32)
    return pl.pallas_call(
        functools.partial(_fps_body, npoint=npoint),
        out_shape=(out, out, out),
    )(X, Y, Z)


def _bq_body(p3_ref, c_ref, idx_ref, *, ns, r2, N):
    b = pl.program_id(0)
    P3 = p3_ref[0]
    C = c_ref[...]
    SB = C.shape[0]
    pp2 = jnp.sum(P3 * P3, axis=0, keepdims=True)
    cc2 = jnp.sum(C * C, axis=1, keepdims=True)
    cross = _dot(C, P3)
    d2 = (cc2 + pp2) - 2.0 * cross
    jidx = lax.broadcasted_iota(jnp.int32, (SB, N), 1)
    klane = lax.broadcasted_iota(jnp.int32, (SB, ns), 1)
    work0 = jnp.where(d2 <= r2, jidx, N)

    def step(k, carry):
        work, IDX = carry
        cur = jnp.min(work, axis=1, keepdims=True)
        IDX = jnp.where(klane == k, cur, IDX)
        work = jnp.where(work == cur, N, work)
        return work, IDX

    _, IDX = lax.fori_loop(0, ns, step, (work0, jnp.full((SB, ns), N, jnp.int32)))
    first = IDX[:, 0:1]
    IDX = jnp.where(IDX < N, IDX, first)
    idx_ref[0] = IDX + b * N


def _ball_query(P3, Cf, S, ns, radius, sblk):
    B = P3.shape[0]
    N = P3.shape[2]
    nsc = S // sblk
    grid = (B, nsc)
    return pl.pallas_call(
        functools.partial(_bq_body, ns=ns, r2=np.float32(radius * radius), N=N),
        grid=grid,
        in_specs=[
            pl.BlockSpec((1, 3, N), lambda b, s: (b, 0, 0)),
            pl.BlockSpec((sblk, 3), lambda b, s: (b * nsc + s, 0)),
        ],
        out_specs=pl.BlockSpec((1, sblk, ns), lambda b, s: (b, s, 0)),
        out_shape=jax.ShapeDtypeStruct((B, S, ns), jnp.int32),
    )(P3, Cf)


def _sc_gather(table, idx, tc_tiling=True):
    V, D = table.shape
    Bi = idx.shape[0]
    info = plsc.get_sparse_core_info()
    NC, NS = info.num_cores, info.num_subcores
    NW = NC * NS
    CH = 128
    per_w = Bi // NW
    n_iter = per_w // CH
    mesh = plsc.VectorSubcoreMesh(core_axis_name="c", subcore_axis_name="s")

    @functools.partial(
        pl.kernel,
        mesh=mesh,
        compiler_params=pltpu.CompilerParams(use_tc_tiling_on_sc=tc_tiling),
        out_type=jax.ShapeDtypeStruct((Bi, D), _F32),
        scratch_types=[
            pltpu.VMEM((2, CH), jnp.int32),
            pltpu.VMEM((2, CH, D), _F32),
            pltpu.SemaphoreType.DMA,
            pltpu.SemaphoreType.DMA,
        ],
    )
    def k(table_hbm, idx_hbm, out_hbm, idx_v, rows_v, sem0, sem1):
        wid = lax.axis_index("s") * NC + lax.axis_index("c")
        base = wid * per_w
        sems = (sem0, sem1)

        def fetch(c, j):
            off = pl.multiple_of(base + c * CH, CH)
            pltpu.sync_copy(idx_hbm.at[pl.ds(off, CH)], idx_v.at[j])
            pltpu.async_copy(table_hbm.at[idx_v.at[j]], rows_v.at[j], sems[j])

        fetch(0, 0)

        def body(i, carry):
            for j in range(2):
                c = 2 * i + j

                @pl.when(c + 1 < n_iter)
                def _():
                    fetch(c + 1, 1 - j)

                pltpu.make_async_copy(
                    table_hbm.at[idx_v.at[j]], rows_v.at[j], sems[j]).wait()
                off = pl.multiple_of(base + c * CH, CH)
                pltpu.sync_copy(rows_v.at[j], out_hbm.at[pl.ds(off, CH)])
            return carry

        lax.fori_loop(0, n_iter // 2, body, 0)

    return k(table, idx)


def _sagroup_body(g_ref, c_ref, w1_ref, b1_ref, g1_ref, bt1_ref,
                  w2_ref, b2_ref, g2_ref, bt2_ref,
                  w3_ref, b3_ref, g3_ref, bt3_ref, out_ref,
                  *, ns, pw, co):
    G = g_ref[...]
    C = c_ref[...]
    RB = C.shape[0]
    crep = jnp.broadcast_to(C[:, None, :], (RB, ns, 3)).reshape(RB * ns, 3)
    rel = G[:, co:co + 3] - crep
    feat = jnp.concatenate([rel, G[:, 0:pw]], axis=1)
    A = _bnrelu(_dot(feat, w1_ref[...]) + b1_ref[...], g1_ref, bt1_ref)
    A = _bnrelu(_dot(A, w2_ref[...]) + b2_ref[...], g2_ref, bt2_ref)
    A = _bnrelu(_dot(A, w3_ref[...]) + b3_ref[...], g3_ref, bt3_ref)
    C3 = A.shape[1]
    out_ref[...] = jnp.max(A.reshape(RB, ns, C3), axis=1)


def _sa_group(G, C, layers, ns, rb, pw, co):
    S = C.shape[0]
    D = G.shape[1]
    (W1, b1, g1, bt1), (W2, b2, g2, bt2), (W3, b3, g3, bt3) = layers
    w1 = W1.T
    w2 = W2.T
    w3 = W3.T
    C3 = W3.shape[0]
    grid = (S // rb,)
    cst = lambda a: pl.BlockSpec(a.shape, lambda i: (0, 0))
    row = lambda a: pl.BlockSpec((1, a.shape[0]), lambda i: (0, 0))
    args = [G, C, w1, b1, g1, bt1, w2, b2, g2, bt2, w3, b3, g3, bt3]
    in_specs = [
        pl.BlockSpec((rb * ns, D), lambda i: (i, 0)),
        pl.BlockSpec((rb, 3), lambda i: (i, 0)),
        cst(w1), row(b1), row(g1), row(bt1),
        cst(w2), row(b2), row(g2), row(bt2),
        cst(w3), row(b3), row(g3), row(bt3),
    ]
    args = [a if a.ndim > 1 else a.reshape(1, -1) for a in args]
    return pl.pallas_call(
        functools.partial(_sagroup_body, ns=ns, pw=pw, co=co),
        grid=grid,
        in_specs=in_specs,
        out_specs=pl.BlockSpec((rb, C3), lambda i: (i, 0)),
        out_shape=jax.ShapeDtypeStruct((S, C3), _F32),
    )(*args)


def _sa3_body(x_ref, *rest, npts):
    A = x_ref[...]
    refs = rest[:-1]
    o_ref = rest[-1]
    for li in range(3):
        w_ref, b_ref, g_ref, bt_ref = refs[4 * li:4 * li + 4]
        A = _bnrelu(_dot(A, w_ref[...]) + b_ref[...], g_ref, bt_ref)
    R, Cc = A.shape
    o_ref[...] = jnp.max(A.reshape(R // npts, npts, Cc), axis=1)


def _sa3(x, layers, npts, rb):
    M = x.shape[0]
    C = layers[-1][0].shape[0]
    grid = (M // rb,)
    cst = lambda a: pl.BlockSpec(a.shape, lambda i: (0, 0))
    in_specs = [pl.BlockSpec((rb, x.shape[1]), lambda i: (i, 0))]
    args = [x]
    for (W, b, g, bt) in layers:
        wt = W.T
        in_specs += [cst(wt)] + [pl.BlockSpec((1, W.shape[0]), lambda i: (0, 0))] * 3
        args += [wt, b.reshape(1, -1), g.reshape(1, -1), bt.reshape(1, -1)]
    return pl.pallas_call(
        functools.partial(_sa3_body, npts=npts),
        grid=grid,
        in_specs=in_specs,
        out_specs=pl.BlockSpec((rb // npts, C), lambda i: (i, 0)),
        out_shape=jax.ShapeDtypeStruct((M // npts, C), _F32),
    )(*args)


def _fp3_body(x1_ref, l3_ref, w1_ref, b1_ref, g1_ref, bt1_ref,
              w2_ref, b2_ref, g2_ref, bt2_ref, o_ref):
    x1 = x1_ref[...]
    l3 = jnp.broadcast_to(l3_ref[0], (x1.shape[0], l3_ref.shape[2]))
    feat = jnp.concatenate([x1, l3], axis=1)
    A = _bnrelu(_dot(feat, w1_ref[...]) + b1_ref[...], g1_ref, bt1_ref)
    o_ref[...] = _bnrelu(_dot(A, w2_ref[...]) + b2_ref[...], g2_ref, bt2_ref)


def _fp3(x1, l3, layers, rb):
    (W1, b1, g1, bt1), (W2, b2, g2, bt2) = layers
    M = x1.shape[0]
    grid = (M // rb,)
    cst = lambda a: pl.BlockSpec(a.shape, lambda b: (0, 0))
    rowv = lambda n: pl.BlockSpec((1, n), lambda b: (0, 0))
    return pl.pallas_call(
        _fp3_body,
        grid=grid,
        in_specs=[
            pl.BlockSpec((rb, x1.shape[1]), lambda b: (b, 0)),
            pl.BlockSpec((1, 1, l3.shape[1]), lambda b: (b, 0, 0)),
            cst(W1.T), rowv(W1.shape[0]), rowv(W1.shape[0]), rowv(W1.shape[0]),
            cst(W2.T), rowv(W2.shape[0]), rowv(W2.shape[0]), rowv(W2.shape[0]),
        ],
        out_specs=pl.BlockSpec((rb, W2.shape[0]), lambda b: (b, 0)),
        out_shape=jax.ShapeDtypeStruct((M, W2.shape[0]), _F32),
    )(x1, l3.reshape(l3.shape[0], 1, l3.shape[1]), W1.T,
      b1.reshape(1, -1), g1.reshape(1, -1), bt1.reshape(1, -1),
      W2.T, b2.reshape(1, -1), g2.reshape(1, -1), bt2.reshape(1, -1))


def _fp_body(qc_ref, s3_ref, p1_ref, ps_ref,
             w1_ref, b1_ref, g1_ref, bt1_ref, rest_refs, o_ref,
             *, flags, dup_xyz):
    QC = qc_ref[...]
    S3 = s3_ref[0]
    NQ = QC.shape[0]
    NS = S3.shape[1]
    qq2 = jnp.sum(QC * QC, axis=1, keepdims=True)
    ss2 = jnp.sum(S3 * S3, axis=0, keepdims=True)
    cross = _dot(QC, S3)
    d2 = (qq2 + ss2) - 2.0 * cross
    jidx = lax.broadcasted_iota(jnp.int32, (NQ, NS), 1)

    work = d2
    ids = []
    ds = []
    for _ in range(3):
        m = jnp.min(work, axis=1, keepdims=True)
        i = jnp.min(jnp.where(work == m, jidx, NS), axis=1, keepdims=True)
        ids.append(i)
        ds.append(m)
        work = jnp.where(jidx == i, _BIG, work)
    r1 = 1.0 / (ds[0] + 1e-8)
    r2 = 1.0 / (ds[1] + 1e-8)
    r3 = 1.0 / (ds[2] + 1e-8)
    s = r1 + r2 + r3
    Wmat = (
        jnp.where(jidx == ids[0], r1 / s, 0.0)
        + jnp.where(jidx == ids[1], r2 / s, 0.0)
        + jnp.where(jidx == ids[2], r3 / s, 0.0)
    )
    interp = jnp.dot(Wmat, ps_ref[...], precision=lax.Precision.HIGHEST,
                     preferred_element_type=_F32)
    p1 = p1_ref[...]
    if dup_xyz:
        feat = jnp.concatenate([p1, p1, interp], axis=1)
    else:
        feat = jnp.concatenate([p1, interp], axis=1)
    A = _bnrelu(_dot(feat, w1_ref[...]) + b1_ref[...], g1_ref, bt1_ref)
    for li, (has_bn, has_relu) in enumerate(flags):
        w_ref, b_ref, g_ref, bt_ref = rest_refs[4 * li:4 * li + 4]
        A = _dot(A, w_ref[...]) + b_ref[...]
        if has_bn:
            A = A / _SQ * g_ref[...] + bt_ref[...]
        if has_relu:
            A = jnp.maximum(A, 0.0)
    o_ref[...] = A


def _fp_stage(QC, S3, P1, PS, layer1, rest, qb):
    B = S3.shape[0]
    Ns = S3.shape[2]
    Nq = QC.shape[0] // B
    C1 = P1.shape[1]
    W1, b1, g1, bt1 = layer1
    nq_blocks = Nq // qb
    grid = (B, nq_blocks)
    cst = lambda a: pl.BlockSpec(a.shape, lambda b, q: (0, 0))
    rowv = lambda n: pl.BlockSpec((1, n), lambda b, q: (0, 0))
    qspec = lambda c: pl.BlockSpec((qb, c), lambda b, q: (b * nq_blocks + q, 0))
    in_specs = [qspec(3),
                pl.BlockSpec((1, 3, Ns), lambda b, q: (b, 0, 0)),
                qspec(C1),
                pl.BlockSpec((Ns, PS.shape[1]), lambda b, q: (b, 0)),
                cst(W1.T), rowv(W1.shape[0]), rowv(W1.shape[0]),
                rowv(W1.shape[0])]
    args = [QC, S3, P1, PS, W1.T, b1.reshape(1, -1),
            g1.reshape(1, -1), bt1.reshape(1, -1)]
    flags = []
    for (W, bb, g, bt, has_bn, has_relu) in rest:
        wt = W.T
        in_specs += [cst(wt), rowv(W.shape[0]), rowv(W.shape[0]), rowv(W.shape[0])]
        args += [wt, bb.reshape(1, -1), g.reshape(1, -1), bt.reshape(1, -1)]
        flags.append((has_bn, has_relu))
    Cout = rest[-1][0].shape[0]

    def body(*refs):
        _fp_body(*refs[:8], refs[8:-1], refs[-1], flags=tuple(flags),
                 dup_xyz=(C1 == 3))

    return pl.pallas_call(
        body,
        grid=grid,
        in_specs=in_specs,
        out_specs=pl.BlockSpec((qb, Cout), lambda b, q: (b * nq_blocks + q, 0)),
        out_shape=jax.ShapeDtypeStruct((B * Nq, Cout), _F32),
    )(*args)


def _flat3(CX, CY, CZ):
    return jnp.stack([CX, CY, CZ], axis=-1).reshape(-1, 3)


def kernel(xyz, params):
    B, N, _ = xyz.shape
    X, Y, Z = xyz[..., 0], xyz[..., 1], xyz[..., 2]
    ones = lambda c: (jnp.zeros((c,), _F32), jnp.ones((c,), _F32),
                      jnp.zeros((c,), _F32))

    P3 = jnp.transpose(xyz, (0, 2, 1))

    CX1, CY1, CZ1 = _fps(X, Y, Z, 512)
    c1f = _flat3(CX1, CY1, CZ1)
    idx1 = _ball_query(P3, c1f, 512, 32, 0.2, 128)
    T1 = jnp.pad(xyz.reshape(-1, 3), ((0, 0), (0, 125)))
    G1 = _sc_gather(T1, idx1.reshape(-1))
    l1p = _sa_group(G1, c1f, params['sa1'], 32, 128, pw=3, co=0)

    C1_3 = jnp.stack([CX1, CY1, CZ1], axis=1)
    CX2, CY2, CZ2 = _fps(CX1, CY1, CZ1, 128)
    c2f = _flat3(CX2, CY2, CZ2)
    idx2 = _ball_query(C1_3, c2f, 128, 64, 0.4, 128)
    T2 = jnp.concatenate(
        [l1p, c1f, jnp.zeros((c1f.shape[0], 125), _F32)], axis=1)
    G2 = _sc_gather(T2, idx2.reshape(-1))
    l2p = _sa_group(G2, c2f, params['sa2'], 64, 64, pw=128, co=128)

    x3 = jnp.concatenate([c2f, l2p], axis=1)
    l3 = _sa3(x3, params['sa3'], 128, 1024)

    l2p_new = _fp3(l2p, l3, params['fp3'], 128)

    (Wf1, bf1, gf1, btf1), (Wf2, bf2, gf2, btf2) = params['fp2']
    C2_3 = jnp.stack([CX2, CY2, CZ2], axis=1)
    l1p_new = _fp_stage(
        c1f, C2_3, l1p, l2p_new,
        (Wf1, bf1, gf1, btf1),
        [(Wf2, bf2, gf2, btf2, True, True)],
        qb=512,
    )

    fp1 = params['fp1']
    Wc1, bc1, gc1, btc1 = params['conv1']
    Wc2, bc2 = params['conv2']
    one1 = ones(Wc2.shape[0])
    rest = [
        (fp1[1][0], fp1[1][1], fp1[1][2], fp1[1][3], True, True),
        (fp1[2][0], fp1[2][1], fp1[2][2], fp1[2][3], True, True),
        (Wc1, bc1, gc1, btc1, True, True),
        (Wc2, bc2, one1[1], one1[0], False, False),
    ]
    out = _fp_stage(
        xyz.reshape(-1, 3), C1_3, xyz.reshape(-1, 3), l1p_new,
        (fp1[0][0], fp1[0][1], fp1[0][2], fp1[0][3]),
        rest,
        qb=1024,
    )
    return out.reshape(B, N, -1)
